# baseline (speedup 1.0000x reference)
import functools

import jax
import jax.numpy as jnp
from jax import lax
from jax.experimental import pallas as pl
from jax.experimental.pallas import tpu as pltpu
from jax.experimental.pallas import tpu_sc as plsc

_L = 16
_ROWS = 64
_N = 8192
_CHUNKS = _N // _L
_WORKERS = 32
_ROWS_PER_W = _ROWS // _WORKERS
_BISECT_ITERS = 30
_G = 4
_NGROUPS = _CHUNKS // _G
_PAD = -1e30


def _row_sparsemax(xbuf, r, cand, candix, gmax, obuf):
    pad16 = jnp.full((_L,), _PAD, jnp.float32)
    lane = lax.iota(jnp.int32, _L)

    def p1(i, acc):
        base = i * (_L * 8)
        vs = [xbuf[r, pl.ds(base + u * _L, _L)] for u in range(8)]
        l1 = [jnp.maximum(vs[2 * j], vs[2 * j + 1]) for j in range(4)]
        l2 = [jnp.maximum(l1[0], l1[1]), jnp.maximum(l1[2], l1[3])]
        gmax[pl.ds(i * (2 * _L), _L)] = l2[0]
        gmax[pl.ds(i * (2 * _L) + _L, _L)] = l2[1]
        return jnp.maximum(acc, jnp.maximum(l2[0], l2[1]))

    with jax.named_scope("p1max"):
        m16 = lax.fori_loop(0, _CHUNKS // 8, p1, pad16)
        m = jnp.max(m16)
    thr = m - 1.0

    def p2(g, off16):
        mv = gmax[pl.ds(g * _L, _L)]

        def do(off):
            for u in range(_G):
                el = g * (_G * _L) + u * _L
                v = xbuf[r, pl.ds(el, _L)]
                msk = v > thr
                ones = jnp.where(msk, 1.0, 0.0).astype(jnp.float32)
                c = plsc.cumsum(ones)
                pos = (c.astype(jnp.int32) - 1) + off
                plsc.store_scatter(cand, [pos], v, mask=msk)
                plsc.store_scatter(candix, [pos], lane + el, mask=msk)
                off = off + plsc.all_reduce_population_count(msk)
            return off

        return lax.cond(jnp.any(mv > thr), do, lambda o: o, off16)

    with jax.named_scope("p2compact"):
        off16 = lax.fori_loop(0, _NGROUPS, p2, jnp.zeros((_L,), jnp.int32))
        cnt = jnp.max(off16.astype(jnp.float32)).astype(jnp.int32)
    cand[pl.ds(cnt, _L)] = pad16
    ncv = (cnt + (_L - 1)) >> 4

    def bis(_, lohi):
        lo, hi = lohi
        mid = 0.5 * (lo + hi)

        def acc_body(i, acc):
            v = cand[pl.ds(i * _L, _L)]
            return acc + jnp.maximum(v - mid, 0.0)

        f = jnp.sum(lax.fori_loop(0, ncv, acc_body,
                                  jnp.zeros((_L,), jnp.float32)))
        gt = f > 1.0
        return jnp.where(gt, mid, lo), jnp.where(gt, hi, mid)

    with jax.named_scope("bisect"):
        lo, hi = lax.fori_loop(0, _BISECT_ITERS, bis, (thr, m))
    tg = 0.5 * (lo + hi)

    def p3(i, carry):
        sv, kv = carry
        v = cand[pl.ds(i * _L, _L)]
        msk = v > tg
        return (sv + jnp.where(msk, v, 0.0),
                kv + jnp.where(msk, 1.0, 0.0))

    sv, kv = lax.fori_loop(
        0, ncv, p3,
        (jnp.zeros((_L,), jnp.float32), jnp.zeros((_L,), jnp.float32)))
    num = jnp.broadcast_to(jnp.sum(sv) - 1.0, (_L,))
    den = jnp.broadcast_to(jnp.maximum(jnp.sum(kv), 1.0), (_L,))
    tau16 = num / den

    row16 = jnp.full((_L,), r, jnp.int32)

    def pout(i, _):
        v = cand[pl.ds(i * _L, _L)]
        ix = candix[pl.ds(i * _L, _L)]
        msk = v > tg
        plsc.store_scatter(obuf, [row16, ix],
                           jnp.maximum(v - tau16, 0.0), mask=msk)
        return 0

    with jax.named_scope("pout"):
        lax.fori_loop(0, ncv, pout, 0)


def kernel(input):
    x = input
    mesh = plsc.VectorSubcoreMesh(core_axis_name="c", subcore_axis_name="s")

    @functools.partial(
        pl.kernel,
        mesh=mesh,
        out_type=jax.ShapeDtypeStruct((_ROWS, _N), jnp.float32),
        scratch_types=[
            pltpu.VMEM((_ROWS_PER_W, _N), jnp.float32),
            pltpu.VMEM((_ROWS_PER_W, _N), jnp.float32),
            pltpu.VMEM((_N + _L,), jnp.float32),
            pltpu.VMEM((_N + _L,), jnp.int32),
            pltpu.VMEM((_NGROUPS * _L,), jnp.float32),
            pltpu.SemaphoreType.DMA,
            pltpu.SemaphoreType.DMA,
        ],
        compiler_params=pltpu.CompilerParams(needs_layout_passes=False),
    )
    def run(x_hbm, out_hbm, xbuf, obuf, cand, candix, gmax, sem_in, sem_out):
        wid = lax.axis_index("s") * 2 + lax.axis_index("c")
        base = wid * _ROWS_PER_W
        cp_in = pltpu.async_copy(x_hbm.at[pl.ds(base, _ROWS_PER_W)],
                                 xbuf, sem_in)

        zero16 = jnp.zeros((_L,), jnp.float32)

        def z(i, _):
            b = i * (_L * 8)
            for rr in range(_ROWS_PER_W):
                for u in range(8):
                    obuf[rr, pl.ds(b + u * _L, _L)] = zero16
            return 0

        with jax.named_scope("zero"):
            lax.fori_loop(0, _CHUNKS // 8, z, 0)
        with jax.named_scope("dmawait"):
            cp_in.wait()

        cps = []
        for r in range(_ROWS_PER_W):
            _row_sparsemax(xbuf, r, cand, candix, gmax, obuf)
            cp = pltpu.async_copy(obuf.at[pl.ds(r, 1)],
                                  out_hbm.at[pl.ds(base + r, 1)], sem_out)
            cps.append(cp)
        for cp in cps:
            cp.wait()

    return run(x)


# speedup vs baseline: 1.0691x; 1.0691x over previous
import functools

import jax
import jax.numpy as jnp
from jax import lax
from jax.experimental import pallas as pl
from jax.experimental.pallas import tpu as pltpu
from jax.experimental.pallas import tpu_sc as plsc

_L = 16
_ROWS = 64
_N = 8192
_CHUNKS = _N // _L
_WORKERS = 32
_ROWS_PER_W = _ROWS // _WORKERS
_BISECT_ITERS = 30
_G = 4
_NGROUPS = _CHUNKS // _G
_PAD = -1e30


def _row_sparsemax(xbuf, r, cand, candix, gmax, obuf):
    pad16 = jnp.full((_L,), _PAD, jnp.float32)
    lane = lax.iota(jnp.int32, _L)

    def p1(i, acc):
        base = i * (_L * 16)
        vs = [xbuf[r, pl.ds(base + u * _L, _L)] for u in range(16)]
        l1 = [jnp.maximum(vs[2 * j], vs[2 * j + 1]) for j in range(8)]
        l2 = [jnp.maximum(l1[2 * j], l1[2 * j + 1]) for j in range(4)]
        for j in range(4):
            gmax[pl.ds(i * (4 * _L) + j * _L, _L)] = l2[j]
        l3 = [jnp.maximum(l2[0], l2[1]), jnp.maximum(l2[2], l2[3])]
        return jnp.maximum(acc, jnp.maximum(l3[0], l3[1]))

    m16 = lax.fori_loop(0, _CHUNKS // 16, p1, pad16)
    m = jnp.max(m16)
    thr = m - 1.0

    def p2(g, off16):
        mv = gmax[pl.ds(g * _L, _L)]

        def do(off):
            for u in range(_G):
                el = g * (_G * _L) + u * _L
                v = xbuf[r, pl.ds(el, _L)]
                msk = v > thr
                ones = jnp.where(msk, 1.0, 0.0).astype(jnp.float32)
                c = plsc.cumsum(ones)
                pos = (c.astype(jnp.int32) - 1) + off
                plsc.store_scatter(cand, [pos], v, mask=msk)
                plsc.store_scatter(candix, [pos], lane + el, mask=msk)
                off = off + plsc.all_reduce_population_count(msk)
            return off

        return lax.cond(jnp.any(mv > thr), do, lambda o: o, off16)

    off16 = lax.fori_loop(0, _NGROUPS, p2, jnp.zeros((_L,), jnp.int32))
    cnt = jnp.max(off16.astype(jnp.float32)).astype(jnp.int32)
    cand[pl.ds(cnt, _L)] = pad16
    ncv = (cnt + (_L - 1)) >> 4

    def mic_cond(state):
        _, k_prev, k_cur, it = state
        return jnp.logical_and(k_cur != k_prev, it < 128)

    def mic_body(state):
        t16, _, k_cur, it = state

        def acc(i, c):
            sv, kv = c
            v = cand[pl.ds(i * _L, _L)]
            msk = v > t16
            return (sv + jnp.where(msk, v, 0.0),
                    kv + jnp.where(msk, 1.0, 0.0))

        sv, kv = lax.fori_loop(
            0, ncv, acc,
            (jnp.zeros((_L,), jnp.float32), jnp.zeros((_L,), jnp.float32)))
        s_tot = jnp.sum(sv)
        k_tot = jnp.sum(kv)
        num = jnp.broadcast_to(s_tot - 1.0, (_L,))
        den = jnp.broadcast_to(jnp.maximum(k_tot, 1.0), (_L,))
        return num / den, k_cur, k_tot, it + 1

    thr16 = jnp.broadcast_to(thr, (_L,))
    tau16, _, _, _ = lax.while_loop(
        mic_cond, mic_body, (thr16, jnp.float32(-1.0), jnp.float32(-2.0),
                             jnp.int32(0)))

    row16 = jnp.full((_L,), r, jnp.int32)

    def pout(i, _):
        v = cand[pl.ds(i * _L, _L)]
        ix = candix[pl.ds(i * _L, _L)]
        msk = v > tau16
        plsc.store_scatter(obuf, [row16, ix],
                           jnp.maximum(v - tau16, 0.0), mask=msk)
        return 0

    lax.fori_loop(0, ncv, pout, 0)


def kernel(input):
    x = input
    mesh = plsc.VectorSubcoreMesh(core_axis_name="c", subcore_axis_name="s")

    @functools.partial(
        pl.kernel,
        mesh=mesh,
        out_type=jax.ShapeDtypeStruct((_ROWS, _N), jnp.float32),
        scratch_types=[
            pltpu.VMEM((_ROWS_PER_W, _N), jnp.float32),
            pltpu.VMEM((_ROWS_PER_W, _N), jnp.float32),
            pltpu.VMEM((_N + _L,), jnp.float32),
            pltpu.VMEM((_N + _L,), jnp.int32),
            pltpu.VMEM((_NGROUPS * _L,), jnp.float32),
            pltpu.SemaphoreType.DMA,
            pltpu.SemaphoreType.DMA,
        ],
        compiler_params=pltpu.CompilerParams(needs_layout_passes=False),
    )
    def run(x_hbm, out_hbm, xbuf, obuf, cand, candix, gmax, sem_in, sem_out):
        wid = lax.axis_index("s") * 2 + lax.axis_index("c")
        base = wid * _ROWS_PER_W
        cp_in = pltpu.async_copy(x_hbm.at[pl.ds(base, _ROWS_PER_W)],
                                 xbuf, sem_in)

        zero16 = jnp.zeros((_L,), jnp.float32)

        def z(i, _):
            b = i * (_L * 8)
            for rr in range(_ROWS_PER_W):
                for u in range(8):
                    obuf[rr, pl.ds(b + u * _L, _L)] = zero16
            return 0

        lax.fori_loop(0, _CHUNKS // 8, z, 0)
        cp_in.wait()

        cps = []
        for r in range(_ROWS_PER_W):
            _row_sparsemax(xbuf, r, cand, candix, gmax, obuf)
            cp = pltpu.async_copy(obuf.at[pl.ds(r, 1)],
                                  out_hbm.at[pl.ds(base + r, 1)], sem_out)
            cps.append(cp)
        for cp in cps:
            cp.wait()

    return run(x)


# speedup vs baseline: 1.2045x; 1.1267x over previous
import functools

import jax
import jax.numpy as jnp
from jax import lax
from jax.experimental import pallas as pl
from jax.experimental.pallas import tpu as pltpu
from jax.experimental.pallas import tpu_sc as plsc

_L = 16
_ROWS = 64
_N = 8192
_CHUNKS = _N // _L
_WORKERS = 32
_RPW = _ROWS // _WORKERS
_G = 4
_NGROUPS = _CHUNKS // _G
_PAD = -1e30


def kernel(input):
    x = input
    mesh = plsc.VectorSubcoreMesh(core_axis_name="c", subcore_axis_name="s")

    @functools.partial(
        pl.kernel,
        mesh=mesh,
        out_type=jax.ShapeDtypeStruct((_ROWS, _N), jnp.float32),
        scratch_types=[
            pltpu.VMEM((_RPW, _N), jnp.float32),
            pltpu.VMEM((_RPW, _N), jnp.float32),
            pltpu.VMEM((_RPW, _N + _L), jnp.float32),
            pltpu.VMEM((_RPW, _N + _L), jnp.int32),
            pltpu.VMEM((_RPW, _NGROUPS * _L), jnp.float32),
            pltpu.SemaphoreType.DMA,
            pltpu.SemaphoreType.DMA,
        ],
        compiler_params=pltpu.CompilerParams(needs_layout_passes=False),
    )
    def run(x_hbm, out_hbm, xbuf, obuf, cand, candix, gmax, sem_in, sem_out):
        wid = lax.axis_index("s") * 2 + lax.axis_index("c")
        base = wid * _RPW
        cp_in = pltpu.async_copy(x_hbm.at[pl.ds(base, _RPW)], xbuf, sem_in)

        zero16 = jnp.zeros((_L,), jnp.float32)

        def z(i, _):
            b = i * (_L * 8)
            for rr in range(_RPW):
                for u in range(8):
                    obuf[rr, pl.ds(b + u * _L, _L)] = zero16
            return 0

        lax.fori_loop(0, _CHUNKS // 8, z, 0)
        cp_in.wait()

        pad16 = jnp.full((_L,), _PAD, jnp.float32)
        lane = lax.iota(jnp.int32, _L)

        def p1(i, carry):
            a0, a1, b0, b1 = carry
            accs = [a0, a1]
            acc2s = [b0, b1]
            out = []
            for r in range(_RPW):
                bel = i * (_L * 8)
                vs = [xbuf[r, pl.ds(bel + u * _L, _L)] for u in range(8)]
                l1 = [jnp.maximum(vs[2 * j], vs[2 * j + 1]) for j in range(4)]
                l2 = [jnp.maximum(l1[0], l1[1]), jnp.maximum(l1[2], l1[3])]
                gmax[r, pl.ds(i * (2 * _L), _L)] = l2[0]
                gmax[r, pl.ds(i * (2 * _L) + _L, _L)] = l2[1]
                t = jnp.maximum(l2[0], l2[1])
                out.append(jnp.maximum(accs[r], t))
                acc2s[r] = jnp.maximum(acc2s[r], jnp.minimum(accs[r], t))
            return out[0], out[1], acc2s[0], acc2s[1]

        m16_0, m16_1, s16_0, s16_1 = lax.fori_loop(
            0, _CHUNKS // 8, p1, (pad16, pad16, pad16, pad16))

        thrs, ms = [], []
        for m16, s16 in ((m16_0, s16_0), (m16_1, s16_1)):
            m = jnp.max(m16)
            m2 = jnp.maximum(
                jnp.max(jnp.where(m16 == m, _PAD, m16)), jnp.max(s16))
            thrs.append(jnp.maximum(m - 1.0, 0.5 * (m + m2 - 1.0)))
            ms.append(m)

        def p2(g, carry):
            hit0 = gmax[0, pl.ds(g * _L, _L)] > thrs[0]
            hit1 = gmax[1, pl.ds(g * _L, _L)] > thrs[1]

            def do(c):
                offs = list(c)
                for u in range(_G):
                    el = g * (_G * _L) + u * _L
                    for r in range(_RPW):
                        v = xbuf[r, pl.ds(el, _L)]
                        msk = v > thrs[r]
                        ones = jnp.where(msk, 1.0, 0.0).astype(jnp.float32)
                        cum = plsc.cumsum(ones)
                        pos = (cum.astype(jnp.int32) - 1) + offs[r]
                        row16 = jnp.full((_L,), r, jnp.int32)
                        plsc.store_scatter(cand, [row16, pos], v, mask=msk)
                        plsc.store_scatter(candix, [row16, pos], lane + el,
                                           mask=msk)
                        offs[r] = offs[r] + \
                            plsc.all_reduce_population_count(msk)
                return tuple(offs)

            return lax.cond(jnp.any(jnp.logical_or(hit0, hit1)), do,
                            lambda c: c, carry)

        zero16i = jnp.zeros((_L,), jnp.int32)
        offs = lax.fori_loop(0, _NGROUPS, p2, (zero16i,) * _RPW)

        for r in range(_RPW):
            cnt = jnp.max(offs[r].astype(jnp.float32)).astype(jnp.int32)
            cand[r, pl.ds(cnt, _L)] = pad16
            ncv = (cnt + (_L - 1)) >> 4

            def mic_cond(state):
                _, k_prev, k_cur, it = state
                return jnp.logical_and(k_cur != k_prev, it < 128)

            def mic_body(state):
                t16, _, k_cur, it = state

                def acc(i, c):
                    sv, kv = c
                    v = cand[r, pl.ds(i * _L, _L)]
                    msk = v > t16
                    return (sv + jnp.where(msk, v, 0.0),
                            kv + jnp.where(msk, 1.0, 0.0))

                sv, kv = lax.fori_loop(
                    0, ncv, acc,
                    (jnp.zeros((_L,), jnp.float32),
                     jnp.zeros((_L,), jnp.float32)))
                s_tot = jnp.sum(sv)
                k_tot = jnp.sum(kv)
                num = jnp.broadcast_to(s_tot - 1.0, (_L,))
                den = jnp.broadcast_to(jnp.maximum(k_tot, 1.0), (_L,))
                return num / den, k_cur, k_tot, it + 1

            thr16 = jnp.broadcast_to(thrs[r], (_L,))
            tau16, _, _, _ = lax.while_loop(
                mic_cond, mic_body,
                (thr16, jnp.float32(-1.0), jnp.float32(-2.0), jnp.int32(0)))

            row16 = jnp.full((_L,), r, jnp.int32)

            def pout(i, _):
                v = cand[r, pl.ds(i * _L, _L)]
                ix = candix[r, pl.ds(i * _L, _L)]
                msk = v > tau16
                plsc.store_scatter(obuf, [row16, ix],
                                   jnp.maximum(v - tau16, 0.0), mask=msk)
                return 0

            lax.fori_loop(0, ncv, pout, 0)

        cps = []
        for r in range(_RPW):
            cps.append(pltpu.async_copy(obuf.at[pl.ds(r, 1)],
                                        out_hbm.at[pl.ds(base + r, 1)],
                                        sem_out))
        for cp in cps:
            cp.wait()

    return run(x)
